# baseline (speedup 1.0000x reference)
import jax
import jax.numpy as jnp
from jax import lax
from jax.experimental import pallas as pl
from jax.experimental.pallas import tpu as pltpu, tpu_sc as plsc

B = 4096
LU = 20
LM = 50
E = 12
EP = 16
NC, NS = 2, 16
NW = NC * NS
BPW = B // NW
CB = 8
NCHUNK = BPW // CB
U_ROWS = CB * LU
M_ROWS = CB * LM
OUT_W = CB * LU * LM


def _body(user1, media1, ut, mt, out_hbm,
          idxu0, idxu1, idxm0, idxm1, uer0, uer1, mer0, mer1, out_buf,
          sg0, sg1):
    wid = lax.axis_index("s") * NC + lax.axis_index("c")
    iota = lax.iota(jnp.int32, 16)
    ecols = [jnp.broadcast_to(jnp.int32(e), (16,)) for e in range(E)]

    def fire(c, idxu, idxm, uer, mer, sg):
        ub0 = wid * (BPW * LU) + c * U_ROWS
        mb0 = wid * (BPW * LM) + c * M_ROWS
        pltpu.sync_copy(user1.at[pl.ds(ub0, U_ROWS)], idxu)
        pltpu.sync_copy(media1.at[pl.ds(mb0, M_ROWS)], idxm)
        for r in range(2):
            pltpu.async_copy(ut.at[idxu.at[pl.ds(r * 80, 80)]],
                             uer.at[pl.ds(r * 80, 80)], sg)
        for r in range(5):
            pltpu.async_copy(mt.at[idxm.at[pl.ds(r * 80, 80)]],
                             mer.at[pl.ds(r * 80, 80)], sg)

    def drain(uer, mer, sg):
        pltpu.make_async_copy(ut.at[pl.ds(0, U_ROWS)], uer, sg).wait()
        pltpu.make_async_copy(mt.at[pl.ds(0, M_ROWS)],
                              mer.at[pl.ds(0, M_ROWS)], sg).wait()

    def compute(c, uer, mer):
        def batch(b, carry):
            ub = b * LU
            mb = b * LM
            for ib in range(4):
                urows = [uer[ub + ib * 5 + ii, :] for ii in range(5)]
                accs = [[None] * 4 for _ in range(5)]
                for e in range(E):
                    mv = [plsc.load_gather(mer, [iota + (mb + jc * 16),
                                                 ecols[e]])
                          for jc in range(4)]
                    for ii in range(5):
                        s = urows[ii][e]
                        for jc in range(4):
                            p = s * mv[jc]
                            accs[ii][jc] = p if e == 0 else accs[ii][jc] + p
                for ii in range(5):
                    i = ib * 5 + ii
                    for jc in range(4):
                        v = 1.0 / (1.0 + jnp.exp(-accs[ii][jc]))
                        out_buf[pl.ds(b * (LU * LM) + i * LM + jc * 16, 16)] = v
            return carry

        lax.fori_loop(0, CB, batch, 0)
        base = (wid * NCHUNK + c) * OUT_W
        pltpu.sync_copy(out_buf.at[pl.ds(0, OUT_W)],
                        out_hbm.at[pl.ds(base, OUT_W)])

    fire(0, idxu0, idxm0, uer0, mer0, sg0)

    def step(k, carry):
        a = 2 * k
        fire(a + 1, idxu1, idxm1, uer1, mer1, sg1)
        drain(uer0, mer0, sg0)
        compute(a, uer0, mer0)

        @pl.when(k < NCHUNK // 2 - 1)
        def _():
            fire(a + 2, idxu0, idxm0, uer0, mer0, sg0)

        drain(uer1, mer1, sg1)
        compute(a + 1, uer1, mer1)
        return carry

    lax.fori_loop(0, NCHUNK // 2, step, 0)


@jax.jit
def kernel(user, media, user_table, media_table):
    user1 = user.astype(jnp.int32).reshape(B * LU)
    media1 = media.astype(jnp.int32).reshape(B * LM)
    mesh = plsc.VectorSubcoreMesh(core_axis_name="c", subcore_axis_name="s",
                                  num_cores=NC, num_subcores=NS)
    out = pl.kernel(
        _body,
        out_type=jax.ShapeDtypeStruct((B * LU * LM,), jnp.float32),
        mesh=mesh,
        scratch_types=[
            pltpu.VMEM((U_ROWS,), jnp.int32),
            pltpu.VMEM((U_ROWS,), jnp.int32),
            pltpu.VMEM((M_ROWS,), jnp.int32),
            pltpu.VMEM((M_ROWS,), jnp.int32),
            pltpu.VMEM((U_ROWS, EP), jnp.float32),
            pltpu.VMEM((U_ROWS, EP), jnp.float32),
            pltpu.VMEM((M_ROWS + 16, EP), jnp.float32),
            pltpu.VMEM((M_ROWS + 16, EP), jnp.float32),
            pltpu.VMEM((OUT_W + 16,), jnp.float32),
            pltpu.SemaphoreType.DMA,
            pltpu.SemaphoreType.DMA,
        ],
        compiler_params=pltpu.CompilerParams(needs_layout_passes=False,
                                             use_tc_tiling_on_sc=False,
                                             disable_bounds_checks=True),
    )(user1, media1,
      jnp.pad(user_table, ((0, 0), (0, EP - E))),
      jnp.pad(media_table, ((0, 0), (0, EP - E))))
    return out.reshape(B, LU, LM)
